# baseline (speedup 1.0000x reference)
import jax
import jax.numpy as jnp
from jax import lax
from jax.experimental import pallas as pl
from jax.experimental.pallas import tpu as pltpu
from jax.experimental.pallas import tpu_sc as plsc

_N, _E, _D = 10000, 320000, 128
_NC, _NS = 2, 16
_NW = _NC * _NS
_EPW = _E // _NW
_C = 80
_NCHUNK = 128
_CAP = _NCHUNK * _C
_RPT = 632
_NPAD = _NS * _RPT
_CNT_PAD = _NS * 640
_F32 = jnp.float32


def _build_agg(with_count):
    mesh = plsc.VectorSubcoreMesh(core_axis_name="c", subcore_axis_name="s")
    out_type = [jax.ShapeDtypeStruct((_NC, _NPAD, _D), _F32)]
    scratch = [
        pltpu.VMEM((_C,), jnp.int32),
        pltpu.VMEM((_C,), jnp.int32),
        pltpu.VMEM((_C, _D), _F32),
        pltpu.VMEM_SHARED((_NPAD, _D), _F32),
        pltpu.SemaphoreType.DMA,
    ]
    if with_count:
        out_type.append(jax.ShapeDtypeStruct((_NC, _CNT_PAD), _F32))
        scratch += [
            pltpu.VMEM((_C,), _F32),
            pltpu.VMEM((640,), _F32),
            pltpu.VMEM_SHARED((_CNT_PAD,), _F32),
        ]

    def body(y_hbm, src_hbm, dst_hbm, agg_out, *rest):
        if with_count:
            cnt_out, src_v, dst_v, rows_v, acc_sp, sem, ones_v, zc_v, cnt_sp = rest
        else:
            src_v, dst_v, rows_v, acc_sp, sem = rest

        core = lax.axis_index("c")
        sub = lax.axis_index("s")
        wid = core * _NS + sub

        @pl.loop(0, _C)
        def _(i):
            for j in range(_D // 16):
                rows_v[i, pl.ds(j * 16, 16)] = jnp.zeros((16,), _F32)

        r0 = sub * _RPT
        nfull = _RPT // _C
        rem = _RPT % _C
        for k in range(nfull):
            pltpu.sync_copy(rows_v, acc_sp.at[pl.ds(r0 + k * _C, _C)])
        if rem:
            pltpu.sync_copy(rows_v.at[pl.ds(0, rem)],
                            acc_sp.at[pl.ds(r0 + nfull * _C, rem)])

        if with_count:
            @pl.loop(0, 640 // 16)
            def _(i):
                zc_v[pl.ds(i * 16, 16)] = jnp.zeros((16,), _F32)

            @pl.loop(0, _C // 16)
            def _(i):
                ones_v[pl.ds(i * 16, 16)] = jnp.ones((16,), _F32)

            pltpu.sync_copy(zc_v, cnt_sp.at[pl.ds(sub * 640, 640)])

        plsc.subcore_barrier()

        ebase = wid * _CAP

        @pl.loop(0, _NCHUNK)
        def _(j):
            b = ebase + j * _C
            pltpu.sync_copy(src_hbm.at[pl.ds(b, _C)], src_v)
            pltpu.sync_copy(dst_hbm.at[pl.ds(b, _C)], dst_v)
            pltpu.async_copy(y_hbm.at[src_v], rows_v, sem).wait()
            pltpu.sync_copy(rows_v, acc_sp.at[dst_v], add=True)
            if with_count:
                pltpu.sync_copy(ones_v, cnt_sp.at[dst_v], add=True)

        plsc.subcore_barrier()

        for k in range(nfull):
            sl = pl.ds(r0 + k * _C, _C)
            pltpu.sync_copy(acc_sp.at[sl], agg_out.at[core].at[sl])
        if rem:
            sl = pl.ds(r0 + nfull * _C, rem)
            pltpu.sync_copy(acc_sp.at[sl], agg_out.at[core].at[sl])
        if with_count:
            sl = pl.ds(sub * 640, 640)
            pltpu.sync_copy(cnt_sp.at[sl], cnt_out.at[core].at[sl])

    return pl.kernel(body, out_type=out_type, mesh=mesh, scratch_types=scratch)


_agg_with_count = _build_agg(True)
_agg_no_count = _build_agg(False)

_BM = 2000


def _sage_tc(xin, p, cnt, Wl, bl, Wr, relu):
    def body(x_ref, p_ref, c_ref, wl_ref, b_ref, wr_ref, o_ref):
        s = p_ref[0] + p_ref[1]
        c = jnp.maximum(c_ref[0] + c_ref[1], 1.0)
        mean = s / c
        acc = lax.dot_general(mean, wl_ref[...], (((1,), (1,)), ((), ())),
                              preferred_element_type=_F32)
        acc = acc + b_ref[...]
        acc = acc + lax.dot_general(x_ref[...], wr_ref[...],
                                    (((1,), (1,)), ((), ())),
                                    preferred_element_type=_F32)
        if relu:
            acc = jnp.maximum(acc, 0.0)
        o_ref[...] = acc

    return pl.pallas_call(
        body,
        grid=(_N // _BM,),
        in_specs=[
            pl.BlockSpec((_BM, _D), lambda i: (i, 0)),
            pl.BlockSpec((_NC, _BM, _D), lambda i: (0, i, 0)),
            pl.BlockSpec((_NC, _BM, 1), lambda i: (0, i, 0)),
            pl.BlockSpec((_D, _D), lambda i: (0, 0)),
            pl.BlockSpec((1, _D), lambda i: (0, 0)),
            pl.BlockSpec((_D, _D), lambda i: (0, 0)),
        ],
        out_specs=pl.BlockSpec((_BM, _D), lambda i: (i, 0)),
        out_shape=jax.ShapeDtypeStruct((_N, _D), _F32),
    )(xin, p, cnt, Wl, bl, Wr)


def _pack_edges(edge_index):
    src = edge_index[0].astype(jnp.int32).reshape(_NW, _EPW)
    dst = edge_index[1].astype(jnp.int32).reshape(_NW, _EPW)
    pad = _CAP - _EPW
    padsrc = (jnp.arange(_NW * pad, dtype=jnp.int32) * 131) % _N
    padrow = _N + (jnp.arange(_NW * pad, dtype=jnp.int32) % (_NPAD - _N))
    srcp = jnp.concatenate([src, padsrc.reshape(_NW, pad)], axis=1)
    dstp = jnp.concatenate([dst, padrow.reshape(_NW, pad)], axis=1)
    return srcp.reshape(_NW * _CAP), dstp.reshape(_NW * _CAP)


def kernel(x, edge_index, Wl0, bl0, Wr0, Wl1, bl1, Wr1):
    src, dst = _pack_edges(edge_index)
    bl0r = bl0.reshape(1, _D)
    bl1r = bl1.reshape(1, _D)

    agg1, cnt_raw = _agg_with_count(x, src, dst)
    cnt = cnt_raw[:, :_N].reshape(_NC, _N, 1)
    h = _sage_tc(x, agg1[:, :_N], cnt, Wl0, bl0r, Wr0, relu=True)
    (agg2,) = _agg_no_count(h, src, dst)
    out = _sage_tc(h, agg2[:, :_N], cnt, Wl1, bl1r, Wr1, relu=False)
    return out


# speedup vs baseline: 1.2416x; 1.2416x over previous
import jax
import jax.numpy as jnp
from jax import lax
from jax.experimental import pallas as pl
from jax.experimental.pallas import tpu as pltpu
from jax.experimental.pallas import tpu_sc as plsc

_N, _E, _D = 10000, 320000, 128
_NC, _NS = 2, 16
_NW = _NC * _NS
_EPW = _E // _NW
_C = 128
_NCHUNK = 80
_CAP = _NCHUNK * _C
_RPT = 632
_NPAD = _NS * _RPT
_CNT_PAD = _NS * 640
_F32 = jnp.float32


def _build_agg(with_count):
    mesh = plsc.VectorSubcoreMesh(core_axis_name="c", subcore_axis_name="s")
    out_type = [jax.ShapeDtypeStruct((_NC, _NPAD, _D), _F32)]
    scratch = [
        pltpu.VMEM((_C,), jnp.int32),
        pltpu.VMEM((_C,), jnp.int32),
        pltpu.VMEM((_C, _D), _F32),
        pltpu.VMEM_SHARED((_NPAD, _D), _F32),
        pltpu.SemaphoreType.DMA,
    ]
    if with_count:
        out_type.append(jax.ShapeDtypeStruct((_NC, _CNT_PAD), _F32))
        scratch += [
            pltpu.VMEM((_C,), _F32),
            pltpu.VMEM((640,), _F32),
            pltpu.VMEM_SHARED((_CNT_PAD,), _F32),
        ]

    def body(y_hbm, src_hbm, dst_hbm, agg_out, *rest):
        if with_count:
            cnt_out, src_v, dst_v, rows_v, acc_sp, sem, ones_v, zc_v, cnt_sp = rest
        else:
            src_v, dst_v, rows_v, acc_sp, sem = rest

        core = lax.axis_index("c")
        sub = lax.axis_index("s")
        wid = core * _NS + sub

        @pl.loop(0, _C)
        def _(i):
            for j in range(_D // 16):
                rows_v[i, pl.ds(j * 16, 16)] = jnp.zeros((16,), _F32)

        r0 = sub * _RPT
        nfull = _RPT // _C
        rem = _RPT % _C
        for k in range(nfull):
            pltpu.sync_copy(rows_v, acc_sp.at[pl.ds(r0 + k * _C, _C)])
        if rem:
            pltpu.sync_copy(rows_v.at[pl.ds(0, rem)],
                            acc_sp.at[pl.ds(r0 + nfull * _C, rem)])

        if with_count:
            @pl.loop(0, 640 // 16)
            def _(i):
                zc_v[pl.ds(i * 16, 16)] = jnp.zeros((16,), _F32)

            @pl.loop(0, _C // 16)
            def _(i):
                ones_v[pl.ds(i * 16, 16)] = jnp.ones((16,), _F32)

            pltpu.sync_copy(zc_v, cnt_sp.at[pl.ds(sub * 640, 640)])

        plsc.subcore_barrier()

        ebase = wid * _CAP

        @pl.loop(0, _NCHUNK)
        def _(j):
            b = ebase + j * _C
            pltpu.sync_copy(src_hbm.at[pl.ds(b, _C)], src_v)
            pltpu.sync_copy(dst_hbm.at[pl.ds(b, _C)], dst_v)
            pltpu.async_copy(y_hbm.at[src_v], rows_v, sem).wait()
            pltpu.sync_copy(rows_v, acc_sp.at[dst_v], add=True)
            if with_count:
                pltpu.sync_copy(ones_v, cnt_sp.at[dst_v], add=True)

        plsc.subcore_barrier()

        for k in range(nfull):
            sl = pl.ds(r0 + k * _C, _C)
            pltpu.sync_copy(acc_sp.at[sl], agg_out.at[core].at[sl])
        if rem:
            sl = pl.ds(r0 + nfull * _C, rem)
            pltpu.sync_copy(acc_sp.at[sl], agg_out.at[core].at[sl])
        if with_count:
            sl = pl.ds(sub * 640, 640)
            pltpu.sync_copy(cnt_sp.at[sl], cnt_out.at[core].at[sl])

    return pl.kernel(body, out_type=out_type, mesh=mesh, scratch_types=scratch)


_agg_with_count = _build_agg(True)
_agg_no_count = _build_agg(False)

_BM = 2000


def _sage_tc(xin, p, cnt, Wl, bl, Wr, relu):
    def body(x_ref, p_ref, c_ref, wl_ref, b_ref, wr_ref, o_ref):
        s = p_ref[0] + p_ref[1]
        c = jnp.maximum(c_ref[0] + c_ref[1], 1.0)
        mean = s / c
        acc = lax.dot_general(mean, wl_ref[...], (((1,), (1,)), ((), ())),
                              preferred_element_type=_F32)
        acc = acc + b_ref[...]
        acc = acc + lax.dot_general(x_ref[...], wr_ref[...],
                                    (((1,), (1,)), ((), ())),
                                    preferred_element_type=_F32)
        if relu:
            acc = jnp.maximum(acc, 0.0)
        o_ref[...] = acc

    return pl.pallas_call(
        body,
        grid=(_N // _BM,),
        in_specs=[
            pl.BlockSpec((_BM, _D), lambda i: (i, 0)),
            pl.BlockSpec((_NC, _BM, _D), lambda i: (0, i, 0)),
            pl.BlockSpec((_NC, _BM, 1), lambda i: (0, i, 0)),
            pl.BlockSpec((_D, _D), lambda i: (0, 0)),
            pl.BlockSpec((1, _D), lambda i: (0, 0)),
            pl.BlockSpec((_D, _D), lambda i: (0, 0)),
        ],
        out_specs=pl.BlockSpec((_BM, _D), lambda i: (i, 0)),
        out_shape=jax.ShapeDtypeStruct((_N, _D), _F32),
    )(xin, p, cnt, Wl, bl, Wr)


def _pack_edges(edge_index):
    src = edge_index[0].astype(jnp.int32).reshape(_NW, _EPW)
    dst = edge_index[1].astype(jnp.int32).reshape(_NW, _EPW)
    pad = _CAP - _EPW
    padsrc = (jnp.arange(_NW * pad, dtype=jnp.int32) * 131) % _N
    padrow = _N + (jnp.arange(_NW * pad, dtype=jnp.int32) % (_NPAD - _N))
    srcp = jnp.concatenate([src, padsrc.reshape(_NW, pad)], axis=1)
    dstp = jnp.concatenate([dst, padrow.reshape(_NW, pad)], axis=1)
    return srcp.reshape(_NW * _CAP), dstp.reshape(_NW * _CAP)


def kernel(x, edge_index, Wl0, bl0, Wr0, Wl1, bl1, Wr1):
    src, dst = _pack_edges(edge_index)
    bl0r = bl0.reshape(1, _D)
    bl1r = bl1.reshape(1, _D)

    agg1, cnt_raw = _agg_with_count(x, src, dst)
    cnt = cnt_raw[:, :_N].reshape(_NC, _N, 1)
    h = _sage_tc(x, agg1[:, :_N], cnt, Wl0, bl0r, Wr0, relu=True)
    (agg2,) = _agg_no_count(h, src, dst)
    out = _sage_tc(h, agg2[:, :_N], cnt, Wl1, bl1r, Wr1, relu=False)
    return out


# speedup vs baseline: 2.2301x; 1.7962x over previous
import jax
import jax.numpy as jnp
from jax import lax
from jax.experimental import pallas as pl
from jax.experimental.pallas import tpu as pltpu
from jax.experimental.pallas import tpu_sc as plsc

_N, _E, _D = 10000, 320000, 128
_NC, _NS = 2, 16
_NW = _NC * _NS
_EPW = _E // _NW
_C = 128
_NCHUNK = 80
_CAP = _NCHUNK * _C
_RPT = 632
_NPAD = _NS * _RPT
_CNT_PAD = _NS * 640
_F32 = jnp.float32


def _build_agg(with_count):
    mesh = plsc.VectorSubcoreMesh(core_axis_name="c", subcore_axis_name="s")
    out_type = [jax.ShapeDtypeStruct((_NC, _NPAD, _D), _F32)]
    scratch = [
        pltpu.VMEM((_C,), jnp.int32),
        pltpu.VMEM((_C,), jnp.int32),
        pltpu.VMEM((_C,), jnp.int32),
        pltpu.VMEM((_C,), jnp.int32),
        pltpu.VMEM((_C, _D), _F32),
        pltpu.VMEM((_C, _D), _F32),
        pltpu.VMEM_SHARED((_NPAD, _D), _F32),
        pltpu.SemaphoreType.DMA,
        pltpu.SemaphoreType.DMA,
        pltpu.SemaphoreType.DMA,
        pltpu.SemaphoreType.DMA,
    ]
    if with_count:
        out_type.append(jax.ShapeDtypeStruct((_NC, _CNT_PAD), _F32))
        scratch += [
            pltpu.VMEM((_C,), _F32),
            pltpu.VMEM((640,), _F32),
            pltpu.VMEM_SHARED((_CNT_PAD,), _F32),
        ]

    def body(y_hbm, src_hbm, dst_hbm, agg_out, *rest):
        if with_count:
            (cnt_out, s0v, s1v, d0v, d1v, r0v, r1v, acc_sp,
             si0, si1, sg0, sg1, ones_v, zc_v, cnt_sp) = rest
        else:
            (s0v, s1v, d0v, d1v, r0v, r1v, acc_sp,
             si0, si1, sg0, sg1) = rest
        srcs, dsts = (s0v, s1v), (d0v, d1v)
        rows, sgs, sis = (r0v, r1v), (sg0, sg1), (si0, si1)
        rows_v = r0v

        core = lax.axis_index("c")
        sub = lax.axis_index("s")
        wid = core * _NS + sub

        @pl.loop(0, _C)
        def _(i):
            for j in range(_D // 16):
                rows_v[i, pl.ds(j * 16, 16)] = jnp.zeros((16,), _F32)

        r0 = sub * _RPT
        nfull = _RPT // _C
        rem = _RPT % _C
        for k in range(nfull):
            pltpu.sync_copy(rows_v, acc_sp.at[pl.ds(r0 + k * _C, _C)])
        if rem:
            pltpu.sync_copy(rows_v.at[pl.ds(0, rem)],
                            acc_sp.at[pl.ds(r0 + nfull * _C, rem)])

        if with_count:
            @pl.loop(0, 640 // 16)
            def _(i):
                zc_v[pl.ds(i * 16, 16)] = jnp.zeros((16,), _F32)

            @pl.loop(0, _C // 16)
            def _(i):
                ones_v[pl.ds(i * 16, 16)] = jnp.ones((16,), _F32)

            pltpu.sync_copy(zc_v, cnt_sp.at[pl.ds(sub * 640, 640)])

        plsc.subcore_barrier()

        ebase = wid * _CAP

        pltpu.async_copy(src_hbm.at[pl.ds(ebase, _C)], s0v, si0)
        pltpu.async_copy(dst_hbm.at[pl.ds(ebase, _C)], d0v, si0)
        pltpu.make_async_copy(src_hbm.at[pl.ds(ebase, _C)], s0v, si0).wait()
        pltpu.make_async_copy(dst_hbm.at[pl.ds(ebase, _C)], d0v, si0).wait()
        pltpu.async_copy(src_hbm.at[pl.ds(ebase + _C, _C)], s1v, si1)
        pltpu.async_copy(dst_hbm.at[pl.ds(ebase + _C, _C)], d1v, si1)
        pltpu.async_copy(y_hbm.at[s0v], r0v, sg0)

        @pl.loop(0, _NCHUNK, step=2)
        def _(j):
            for p in range(2):
                jj = j + p
                q = 1 - p
                b = ebase + jj * _C

                @pl.when(jj + 1 < _NCHUNK)
                def _():
                    bn = b + _C
                    pltpu.make_async_copy(src_hbm.at[pl.ds(bn, _C)],
                                          srcs[q], sis[q]).wait()
                    pltpu.make_async_copy(dst_hbm.at[pl.ds(bn, _C)],
                                          dsts[q], sis[q]).wait()
                    pltpu.async_copy(y_hbm.at[srcs[q]], rows[q], sgs[q])

                pltpu.make_async_copy(y_hbm.at[srcs[p]],
                                      rows[p], sgs[p]).wait()
                pltpu.sync_copy(rows[p], acc_sp.at[dsts[p]], add=True)
                if with_count:
                    pltpu.sync_copy(ones_v, cnt_sp.at[dsts[p]], add=True)

                @pl.when(jj + 2 < _NCHUNK)
                def _():
                    b2 = b + 2 * _C
                    pltpu.async_copy(src_hbm.at[pl.ds(b2, _C)],
                                     srcs[p], sis[p])
                    pltpu.async_copy(dst_hbm.at[pl.ds(b2, _C)],
                                     dsts[p], sis[p])

        plsc.subcore_barrier()

        for k in range(nfull):
            sl = pl.ds(r0 + k * _C, _C)
            pltpu.sync_copy(acc_sp.at[sl], agg_out.at[core].at[sl])
        if rem:
            sl = pl.ds(r0 + nfull * _C, rem)
            pltpu.sync_copy(acc_sp.at[sl], agg_out.at[core].at[sl])
        if with_count:
            sl = pl.ds(sub * 640, 640)
            pltpu.sync_copy(cnt_sp.at[sl], cnt_out.at[core].at[sl])

    return pl.kernel(body, out_type=out_type, mesh=mesh, scratch_types=scratch)


_agg_with_count = _build_agg(True)
_agg_no_count = _build_agg(False)

_BM = 2000


def _sage_tc(xin, p, cnt, Wl, bl, Wr, relu):
    def body(x_ref, p_ref, c_ref, wl_ref, b_ref, wr_ref, o_ref):
        s = p_ref[0] + p_ref[1]
        c = jnp.maximum(c_ref[0] + c_ref[1], 1.0)
        mean = s / c
        acc = lax.dot_general(mean, wl_ref[...], (((1,), (1,)), ((), ())),
                              preferred_element_type=_F32)
        acc = acc + b_ref[...]
        acc = acc + lax.dot_general(x_ref[...], wr_ref[...],
                                    (((1,), (1,)), ((), ())),
                                    preferred_element_type=_F32)
        if relu:
            acc = jnp.maximum(acc, 0.0)
        o_ref[...] = acc

    return pl.pallas_call(
        body,
        grid=(_N // _BM,),
        in_specs=[
            pl.BlockSpec((_BM, _D), lambda i: (i, 0)),
            pl.BlockSpec((_NC, _BM, _D), lambda i: (0, i, 0)),
            pl.BlockSpec((_NC, _BM, 1), lambda i: (0, i, 0)),
            pl.BlockSpec((_D, _D), lambda i: (0, 0)),
            pl.BlockSpec((1, _D), lambda i: (0, 0)),
            pl.BlockSpec((_D, _D), lambda i: (0, 0)),
        ],
        out_specs=pl.BlockSpec((_BM, _D), lambda i: (i, 0)),
        out_shape=jax.ShapeDtypeStruct((_N, _D), _F32),
    )(xin, p, cnt, Wl, bl, Wr)


def _pack_edges(edge_index):
    src = edge_index[0].astype(jnp.int32).reshape(_NW, _EPW)
    dst = edge_index[1].astype(jnp.int32).reshape(_NW, _EPW)
    pad = _CAP - _EPW
    padsrc = (jnp.arange(_NW * pad, dtype=jnp.int32) * 131) % _N
    padrow = _N + (jnp.arange(_NW * pad, dtype=jnp.int32) % (_NPAD - _N))
    srcp = jnp.concatenate([src, padsrc.reshape(_NW, pad)], axis=1)
    dstp = jnp.concatenate([dst, padrow.reshape(_NW, pad)], axis=1)
    return srcp.reshape(_NW * _CAP), dstp.reshape(_NW * _CAP)


def kernel(x, edge_index, Wl0, bl0, Wr0, Wl1, bl1, Wr1):
    src, dst = _pack_edges(edge_index)
    bl0r = bl0.reshape(1, _D)
    bl1r = bl1.reshape(1, _D)

    agg1, cnt_raw = _agg_with_count(x, src, dst)
    cnt = cnt_raw[:, :_N].reshape(_NC, _N, 1)
    h = _sage_tc(x, agg1[:, :_N], cnt, Wl0, bl0r, Wr0, relu=True)
    (agg2,) = _agg_no_count(h, src, dst)
    out = _sage_tc(h, agg2[:, :_N], cnt, Wl1, bl1r, Wr1, relu=False)
    return out


# speedup vs baseline: 2.4851x; 1.1143x over previous
import jax
import jax.numpy as jnp
from jax import lax
from jax.experimental import pallas as pl
from jax.experimental.pallas import tpu as pltpu
from jax.experimental.pallas import tpu_sc as plsc

_N, _E, _D = 10000, 320000, 128
_NC, _NS = 2, 16
_NW = _NC * _NS
_EPW = _E // _NW
_C = 128
_NCHUNK = 80
_CAP = _NCHUNK * _C
_RPT = 632
_NPAD = _NS * _RPT
_CNT_PAD = _NS * 640
_F32 = jnp.float32


def _build_agg(with_count):
    mesh = plsc.VectorSubcoreMesh(core_axis_name="c", subcore_axis_name="s")
    out_type = [jax.ShapeDtypeStruct((_NC, _NPAD, _D), _F32)]
    scratch = (
        [pltpu.VMEM((_C,), jnp.int32) for _ in range(4)] +
        [pltpu.VMEM((_C,), jnp.int32) for _ in range(4)] +
        [pltpu.VMEM((_C, _D), _F32) for _ in range(2)] +
        [pltpu.VMEM_SHARED((_NPAD, _D), _F32)] +
        [pltpu.SemaphoreType.DMA for _ in range(4)] +
        [pltpu.SemaphoreType.DMA for _ in range(2)] +
        [pltpu.SemaphoreType.DMA for _ in range(2)]
    )
    if with_count:
        out_type.append(jax.ShapeDtypeStruct((_NC, _CNT_PAD), _F32))
        scratch += [
            pltpu.VMEM((_C,), _F32),
            pltpu.VMEM((640,), _F32),
            pltpu.VMEM_SHARED((_CNT_PAD,), _F32),
        ]

    def body(y_hbm, src_hbm, dst_hbm, agg_out, *rest):
        if with_count:
            cnt_out = rest[0]
            rest = rest[1:]
        srcs, dsts = rest[0:4], rest[4:8]
        rows = rest[8:10]
        acc_sp = rest[10]
        sis, sgs, sss = rest[11:15], rest[15:17], rest[17:19]
        if with_count:
            ones_v, zc_v, cnt_sp = rest[19:22]
        rows_v = rows[0]

        core = lax.axis_index("c")
        sub = lax.axis_index("s")
        wid = core * _NS + sub

        @pl.loop(0, _C)
        def _(i):
            for j in range(_D // 16):
                rows_v[i, pl.ds(j * 16, 16)] = jnp.zeros((16,), _F32)

        r0 = sub * _RPT
        nfull = _RPT // _C
        rem = _RPT % _C
        for k in range(nfull):
            pltpu.sync_copy(rows_v, acc_sp.at[pl.ds(r0 + k * _C, _C)])
        if rem:
            pltpu.sync_copy(rows_v.at[pl.ds(0, rem)],
                            acc_sp.at[pl.ds(r0 + nfull * _C, rem)])

        if with_count:
            @pl.loop(0, 640 // 16)
            def _(i):
                zc_v[pl.ds(i * 16, 16)] = jnp.zeros((16,), _F32)

            @pl.loop(0, _C // 16)
            def _(i):
                ones_v[pl.ds(i * 16, 16)] = jnp.ones((16,), _F32)

            pltpu.sync_copy(zc_v, cnt_sp.at[pl.ds(sub * 640, 640)])

        plsc.subcore_barrier()

        ebase = wid * _CAP

        for m in range(4):
            bm = ebase + m * _C
            pltpu.async_copy(src_hbm.at[pl.ds(bm, _C)], srcs[m], sis[m])
            pltpu.async_copy(dst_hbm.at[pl.ds(bm, _C)], dsts[m], sis[m])
        pltpu.make_async_copy(src_hbm.at[pl.ds(ebase, _C)],
                              srcs[0], sis[0]).wait()
        pltpu.make_async_copy(dst_hbm.at[pl.ds(ebase, _C)],
                              dsts[0], sis[0]).wait()
        pltpu.async_copy(y_hbm.at[srcs[0]], rows[0], sgs[0])

        @pl.loop(0, _NCHUNK, step=4)
        def _(j):
            for p in range(4):
                jj = j + p
                m = p
                m1 = (p + 1) % 4
                m3 = (p + 3) % 4
                rp = p % 2
                rq = 1 - rp
                b = ebase + jj * _C

                @pl.when(jj >= 1)
                def _():
                    pltpu.make_async_copy(rows[rq], acc_sp.at[dsts[m3]],
                                          sss[rq]).wait()

                    @pl.when(jj + 3 < _NCHUNK)
                    def _():
                        b3 = b + 3 * _C
                        pltpu.async_copy(src_hbm.at[pl.ds(b3, _C)],
                                         srcs[m3], sis[m3])
                        pltpu.async_copy(dst_hbm.at[pl.ds(b3, _C)],
                                         dsts[m3], sis[m3])

                @pl.when(jj + 1 < _NCHUNK)
                def _():
                    bn = b + _C
                    pltpu.make_async_copy(src_hbm.at[pl.ds(bn, _C)],
                                          srcs[m1], sis[m1]).wait()
                    pltpu.make_async_copy(dst_hbm.at[pl.ds(bn, _C)],
                                          dsts[m1], sis[m1]).wait()
                    pltpu.async_copy(y_hbm.at[srcs[m1]], rows[rq], sgs[rq])

                pltpu.make_async_copy(y_hbm.at[srcs[m]],
                                      rows[rp], sgs[rp]).wait()
                pltpu.async_copy(rows[rp], acc_sp.at[dsts[m]],
                                 sss[rp], add=True)
                if with_count:
                    pltpu.sync_copy(ones_v, cnt_sp.at[dsts[m]], add=True)

        pltpu.make_async_copy(rows[(_NCHUNK - 1) % 2],
                              acc_sp.at[dsts[(_NCHUNK - 1) % 4]],
                              sss[(_NCHUNK - 1) % 2]).wait()

        plsc.subcore_barrier()

        for k in range(nfull):
            sl = pl.ds(r0 + k * _C, _C)
            pltpu.sync_copy(acc_sp.at[sl], agg_out.at[core].at[sl])
        if rem:
            sl = pl.ds(r0 + nfull * _C, rem)
            pltpu.sync_copy(acc_sp.at[sl], agg_out.at[core].at[sl])
        if with_count:
            sl = pl.ds(sub * 640, 640)
            pltpu.sync_copy(cnt_sp.at[sl], cnt_out.at[core].at[sl])

    return pl.kernel(body, out_type=out_type, mesh=mesh, scratch_types=scratch)


_agg_with_count = _build_agg(True)
_agg_no_count = _build_agg(False)

_BM = 2000


def _sage_tc(xin, p, cnt, Wl, bl, Wr, relu):
    def body(x_ref, p_ref, c_ref, wl_ref, b_ref, wr_ref, o_ref):
        s = p_ref[0] + p_ref[1]
        c = jnp.maximum(c_ref[0] + c_ref[1], 1.0)
        mean = s / c
        acc = lax.dot_general(mean, wl_ref[...], (((1,), (1,)), ((), ())),
                              preferred_element_type=_F32)
        acc = acc + b_ref[...]
        acc = acc + lax.dot_general(x_ref[...], wr_ref[...],
                                    (((1,), (1,)), ((), ())),
                                    preferred_element_type=_F32)
        if relu:
            acc = jnp.maximum(acc, 0.0)
        o_ref[...] = acc

    return pl.pallas_call(
        body,
        grid=(_N // _BM,),
        in_specs=[
            pl.BlockSpec((_BM, _D), lambda i: (i, 0)),
            pl.BlockSpec((_NC, _BM, _D), lambda i: (0, i, 0)),
            pl.BlockSpec((_NC, _BM, 1), lambda i: (0, i, 0)),
            pl.BlockSpec((_D, _D), lambda i: (0, 0)),
            pl.BlockSpec((1, _D), lambda i: (0, 0)),
            pl.BlockSpec((_D, _D), lambda i: (0, 0)),
        ],
        out_specs=pl.BlockSpec((_BM, _D), lambda i: (i, 0)),
        out_shape=jax.ShapeDtypeStruct((_N, _D), _F32),
    )(xin, p, cnt, Wl, bl, Wr)


def _pack_edges(edge_index):
    src = edge_index[0].astype(jnp.int32).reshape(_NW, _EPW)
    dst = edge_index[1].astype(jnp.int32).reshape(_NW, _EPW)
    pad = _CAP - _EPW
    padsrc = (jnp.arange(_NW * pad, dtype=jnp.int32) * 131) % _N
    padrow = _N + (jnp.arange(_NW * pad, dtype=jnp.int32) % (_NPAD - _N))
    srcp = jnp.concatenate([src, padsrc.reshape(_NW, pad)], axis=1)
    dstp = jnp.concatenate([dst, padrow.reshape(_NW, pad)], axis=1)
    return srcp.reshape(_NW * _CAP), dstp.reshape(_NW * _CAP)


def kernel(x, edge_index, Wl0, bl0, Wr0, Wl1, bl1, Wr1):
    src, dst = _pack_edges(edge_index)
    bl0r = bl0.reshape(1, _D)
    bl1r = bl1.reshape(1, _D)

    agg1, cnt_raw = _agg_with_count(x, src, dst)
    cnt = cnt_raw[:, :_N].reshape(_NC, _N, 1)
    h = _sage_tc(x, agg1[:, :_N], cnt, Wl0, bl0r, Wr0, relu=True)
    (agg2,) = _agg_no_count(h, src, dst)
    out = _sage_tc(h, agg2[:, :_N], cnt, Wl1, bl1r, Wr1, relu=False)
    return out
